# baseline (speedup 1.0000x reference)
import functools

import jax
import jax.numpy as jnp
import numpy as np
from jax import lax
from jax.experimental import pallas as pl
from jax.experimental.pallas import tpu as pltpu
from jax.experimental.pallas import tpu_sc as plsc


def _pairs():
    arg1s = [[8, 9], [17, 18], [26, 27]]
    arg2s = [[11, 12, 13, 14, 15, 16], [20, 21, 22, 23, 24, 25],
             [29, 30, 31, 32, 33, 34]]
    prods = []
    for a, b in zip(arg1s, arg2s):
        for i in a:
            for j in b:
                prods.append((i, j))
    return np.array(prods, dtype=np.int32)


_P = _pairs()

_COLS = np.zeros((2, 3, 16), np.int32)
_COLS[0].flat[:36] = _P[:, 0]
_COLS[1].flat[:36] = _P[:, 1]

_ROWS = 2048 * 512
_NC, _NS = 2, 16
_NW = _NC * _NS
_RPW = _ROWS // _NW
_CH = 256
_NCHUNK = _RPW // _CH

_mesh = plsc.VectorSubcoreMesh(core_axis_name="c", subcore_axis_name="s")


@functools.partial(
    pl.kernel,
    out_type=jax.ShapeDtypeStruct((_ROWS, 100), jnp.float32),
    mesh=_mesh,
    scratch_types=[
        pltpu.VMEM((2, 3, 16), jnp.int32),
        pltpu.VMEM((_CH, 100), jnp.float32),
    ],
    compiler_params=pltpu.CompilerParams(use_tc_tiling_on_sc=False, needs_layout_passes=False),
)
def _sc_kernel(x_hbm, cols_hbm, out_hbm, ct_v, ov):
    wid = lax.axis_index("s") * _NC + lax.axis_index("c")
    base = wid * _RPW

    pltpu.sync_copy(cols_hbm, ct_v)
    c0 = [ct_v[0, j, :] for j in range(3)]
    c1 = [ct_v[1, j, :] for j in range(3)]
    lanes = lax.iota(jnp.int32, 16)
    tail_cols = 96 + lanes
    tail_mask = lanes < 4

    def chunk_body(ci, carry):
        r0 = base + ci * _CH
        pltpu.sync_copy(x_hbm.at[pl.ds(r0, _CH)], ov.at[:, pl.ds(0, 64)])

        def row_body(r, carry2):
            rowv = jnp.full((16,), r, jnp.int32)
            for j in range(2):
                g0 = plsc.load_gather(ov, [rowv, c0[j]])
                g1 = plsc.load_gather(ov, [rowv, c1[j]])
                ov[r, pl.ds(64 + j * 16, 16)] = g0 * g1
            g0 = plsc.load_gather(ov, [rowv, c0[2]])
            g1 = plsc.load_gather(ov, [rowv, c1[2]])
            plsc.store_scatter(ov, [rowv, tail_cols], g0 * g1,
                               mask=tail_mask)
            return carry2

        lax.fori_loop(0, _CH, row_body, 0)
        pltpu.sync_copy(ov, out_hbm.at[pl.ds(r0, _CH)])
        return carry

    lax.fori_loop(0, _NCHUNK, chunk_body, 0)


@jax.jit
def kernel(x):
    xf = x.reshape(_ROWS, 64)
    out = _sc_kernel(xf, jnp.asarray(_COLS))
    return out.reshape(x.shape[0], x.shape[1], 100)


# speedup vs baseline: 1.0763x; 1.0763x over previous
import functools

import jax
import jax.numpy as jnp
from jax import lax
from jax.experimental import pallas as pl
from jax.experimental.pallas import tpu as pltpu
from jax.experimental.pallas import tpu_sc as plsc

_D0, _D1, _D2 = 2048, 512, 64
_NC, _NS = 2, 16
_NW = _NC * _NS
_SPW = _D0 // _NW

_mesh = plsc.VectorSubcoreMesh(core_axis_name="c", subcore_axis_name="s")


@functools.partial(
    pl.kernel,
    out_type=jax.ShapeDtypeStruct((_D0, _D1, 100), jnp.float32),
    mesh=_mesh,
    scratch_types=[
        pltpu.VMEM((_D1, 100), jnp.float32),
    ],
    compiler_params=pltpu.CompilerParams(use_tc_tiling_on_sc=False,
                                         needs_layout_passes=False),
)
def _sc_kernel(x_hbm, out_hbm, ov):
    wid = lax.axis_index("s") * _NC + lax.axis_index("c")
    base = wid * _SPW

    lanes = lax.iota(jnp.int32, 16)
    c0 = []
    c1 = []
    for j in range(3):
        k = jnp.minimum(j * 16 + lanes, 35)
        g = k // 12
        c0.append(8 + 9 * g + (k // 6) % 2)
        c1.append(11 + 9 * g + k % 6)
    tail_cols = 96 + lanes
    tail_mask = lanes < 4

    def slice_body(si, carry):
        d = base + si
        pltpu.sync_copy(x_hbm.at[d], ov.at[:, pl.ds(0, 64)])

        def row_body(r, carry2):
            rowv = jnp.full((16,), r, jnp.int32)
            for j in range(2):
                g0 = plsc.load_gather(ov, [rowv, c0[j]])
                g1 = plsc.load_gather(ov, [rowv, c1[j]])
                ov[r, pl.ds(64 + j * 16, 16)] = g0 * g1
            g0 = plsc.load_gather(ov, [rowv, c0[2]])
            g1 = plsc.load_gather(ov, [rowv, c1[2]])
            plsc.store_scatter(ov, [rowv, tail_cols], g0 * g1,
                               mask=tail_mask)
            return carry2

        lax.fori_loop(0, _D1, row_body, 0)
        pltpu.sync_copy(ov, out_hbm.at[d])
        return carry

    lax.fori_loop(0, _SPW, slice_body, 0)


@jax.jit
def kernel(x):
    return _sc_kernel(x)


# speedup vs baseline: 8.6327x; 8.0203x over previous
import functools

import jax
import jax.numpy as jnp
import numpy as np
from jax.experimental import pallas as pl
from jax.experimental.pallas import tpu as pltpu


def _pairs():
    arg1s = [[8, 9], [17, 18], [26, 27]]
    arg2s = [[11, 12, 13, 14, 15, 16], [20, 21, 22, 23, 24, 25],
             [29, 30, 31, 32, 33, 34]]
    prods = []
    for a, b in zip(arg1s, arg2s):
        for i in a:
            for j in b:
                prods.append((i, j))
    return np.array(prods, dtype=np.int32)


_P = _pairs()
_D0, _D1, _D2 = 2048, 512, 64
_B0 = 8


def _body(x_ref, o_ref):
    for f in range(_D2):
        o_ref[f] = x_ref[:, f, :]
    for k in range(36):
        o_ref[_D2 + k] = x_ref[:, _P[k, 0], :] * x_ref[:, _P[k, 1], :]


@jax.jit
def kernel(x):
    xt = jnp.transpose(x, (0, 2, 1))
    ot = pl.pallas_call(
        _body,
        grid=(_D0 // _B0,),
        in_specs=[pl.BlockSpec((_B0, _D2, _D1), lambda i: (i, 0, 0))],
        out_specs=pl.BlockSpec((100, _B0, _D1), lambda i: (0, i, 0)),
        out_shape=jax.ShapeDtypeStruct((100, _D0, _D1), jnp.float32),
    )(xt)
    return jnp.transpose(ot, (1, 2, 0))


# speedup vs baseline: 11.9480x; 1.3840x over previous
import functools

import jax
import jax.numpy as jnp
import numpy as np
from jax.experimental import pallas as pl
from jax.experimental.pallas import tpu as pltpu


def _pairs():
    arg1s = [[8, 9], [17, 18], [26, 27]]
    arg2s = [[11, 12, 13, 14, 15, 16], [20, 21, 22, 23, 24, 25],
             [29, 30, 31, 32, 33, 34]]
    prods = []
    for a, b in zip(arg1s, arg2s):
        for i in a:
            for j in b:
                prods.append((i, j))
    return np.array(prods, dtype=np.int32)


_P = _pairs()
_D0, _D1, _D2 = 2048, 512, 64
_B0 = 32


def _body(x_ref, o_ref):
    for f in range(_D2):
        o_ref[f] = x_ref[:, f, :]
    for k in range(36):
        o_ref[_D2 + k] = x_ref[:, _P[k, 0], :] * x_ref[:, _P[k, 1], :]


@jax.jit
def kernel(x):
    xt = jnp.transpose(x, (0, 2, 1))
    ot = pl.pallas_call(
        _body,
        grid=(_D0 // _B0,),
        in_specs=[pl.BlockSpec((_B0, _D2, _D1), lambda i: (i, 0, 0))],
        out_specs=pl.BlockSpec((100, _B0, _D1), lambda i: (0, i, 0)),
        out_shape=jax.ShapeDtypeStruct((100, _D0, _D1), jnp.float32),
    )(xt)
    return jnp.transpose(ot, (1, 2, 0))


# speedup vs baseline: 12.5887x; 1.0536x over previous
import functools

import jax
import jax.numpy as jnp
import numpy as np
from jax.experimental import pallas as pl
from jax.experimental.pallas import tpu as pltpu


def _pairs():
    arg1s = [[8, 9], [17, 18], [26, 27]]
    arg2s = [[11, 12, 13, 14, 15, 16], [20, 21, 22, 23, 24, 25],
             [29, 30, 31, 32, 33, 34]]
    prods = []
    for a, b in zip(arg1s, arg2s):
        for i in a:
            for j in b:
                prods.append((i, j))
    return np.array(prods, dtype=np.int32)


_P = _pairs()
_D0, _D1, _D2 = 2048, 512, 64
_B0 = 64


def _body(x_ref, o_ref):
    for f in range(_D2):
        o_ref[f] = x_ref[:, f, :]
    for k in range(36):
        o_ref[_D2 + k] = x_ref[:, _P[k, 0], :] * x_ref[:, _P[k, 1], :]


@jax.jit
def kernel(x):
    xt = jnp.transpose(x, (0, 2, 1))
    ot = pl.pallas_call(
        _body,
        grid=(_D0 // _B0,),
        in_specs=[pl.BlockSpec((_B0, _D2, _D1), lambda i: (i, 0, 0))],
        out_specs=pl.BlockSpec((100, _B0, _D1), lambda i: (0, i, 0)),
        out_shape=jax.ShapeDtypeStruct((100, _D0, _D1), jnp.float32),
    )(xt)
    return jnp.transpose(ot, (1, 2, 0))
